# baseline (speedup 1.0000x reference)
import jax
import jax.numpy as jnp
from jax.experimental import pallas as pl

_BLOCK_ROWS = 8000


def _body(x_ref, o_ref):
    i = pl.program_id(0)
    v = x_ref[...] + 1.0

    @pl.when(i == 0)
    def _():
        row = jax.lax.broadcasted_iota(jnp.int32, v.shape, 0)
        o_ref[...] = jnp.where(row == 1, jnp.float32(2.0), v)

    @pl.when(i != 0)
    def _():
        o_ref[...] = v


@jax.jit
def kernel(x):
    n, d = x.shape
    return pl.pallas_call(
        _body,
        grid=(n // _BLOCK_ROWS,),
        in_specs=[pl.BlockSpec((_BLOCK_ROWS, d), lambda i: (i, 0))],
        out_specs=pl.BlockSpec((_BLOCK_ROWS, d), lambda i: (i, 0)),
        out_shape=jax.ShapeDtypeStruct((n, d), x.dtype),
    )(x)
